# baseline (speedup 1.0000x reference)
import functools

import jax
import jax.numpy as jnp
from jax import lax
from jax.experimental import pallas as pl
from jax.experimental.pallas import tpu as pltpu
from jax.experimental.pallas import tpu_sc as plsc

B = 4096
L = 200
D = 32
N = B * L

IDX_W = 128
K = 4
CH = K * IDX_W
DR = D // 8

_info = plsc.get_sparse_core_info()
NC, NS = _info.num_cores, _info.num_subcores
NW = NC * NS
NTASK = N // CH
TPW = NTASK // NW
BCHUNKS = B // CH
assert NTASK % NW == 0 and TPW % 2 == 0


def _emb_gather_body(idx_hbm, table_hbm, out_hbm,
                     idx0, idx1, rows0, rows1, tb0, tb1,
                     isem0, isem1, gsem0, gsem1, ssem0, ssem1):
    wid = lax.axis_index("s") * NC + lax.axis_index("c")
    g0 = wid * TPW
    bufs = ((idx0, isem0, rows0, gsem0, tb0, ssem0),
            (idx1, isem1, rows1, gsem1, tb1, ssem1))

    iota16 = lax.iota(jnp.int32, 16)
    dcols = [jnp.full((16,), d, jnp.int32) for d in range(D)]

    def task_lb(t):
        g = g0 + t
        return g // BCHUNKS, (g % BCHUNKS) * K

    def start_idx(t, p):
        idx_v, isem = bufs[p][0], bufs[p][1]
        l, bc0 = task_lb(t)
        pltpu.make_async_copy(
            idx_hbm.at[l, pl.ds(pl.multiple_of(bc0, K), K)], idx_v, isem).start()

    def fire_gathers(p):
        idx_v, _, rows_v, gsem = bufs[p][:4]
        gathers = [
            pltpu.make_async_copy(
                table_hbm.at[idx_v.at[j]],
                rows_v.at[pl.ds(j * IDX_W, IDX_W)],
                gsem,
            )
            for j in range(K)
        ]
        for g in gathers:
            g.start()

    def wait_gathers(p):
        idx_v, _, rows_v, gsem = bufs[p][:4]
        for j in range(K):
            pltpu.make_async_copy(
                table_hbm.at[idx_v.at[j]],
                rows_v.at[pl.ds(j * IDX_W, IDX_W)],
                gsem,
            ).wait()

    def do_task(t, p, first=False, fire_next=True):
        idx_v, isem, rows_v, gsem, tb_v, ssem = bufs[p]
        q = 1 - p
        l, bc0 = task_lb(t)
        wait_gathers(p)
        if fire_next:
            pltpu.make_async_copy(
                idx_hbm.at[0, pl.ds(0, K)], bufs[q][0], bufs[q][1]).wait()
            fire_gathers(q)
            start_idx(jnp.minimum(t + 2, TPW - 1), p)
        if not first:
            pltpu.make_async_copy(
                tb_v, out_hbm.at[pl.ds(0, DR), pl.ds(0, K)], ssem).wait()

        def tr_block(j, _):
            base = j * IDX_W + iota16
            for bg in range(8):
                ridx = base + bg * 16
                for d in range(D):
                    v = plsc.load_gather(rows_v, [ridx, dcols[d]])
                    tb_v[d // 8, j, d % 8, pl.ds(bg * 16, 16)] = v
            return 0

        lax.fori_loop(0, K, tr_block, 0)
        pltpu.make_async_copy(
            tb_v,
            out_hbm.at[pl.ds(pl.multiple_of(l * DR, DR), DR),
                       pl.ds(pl.multiple_of(bc0, K), K)],
            ssem).start()

    start_idx(0, 0)
    pltpu.make_async_copy(idx_hbm.at[0, pl.ds(0, K)], idx0, isem0).wait()
    fire_gathers(0)
    start_idx(1, 1)

    do_task(0, 0, first=True)
    do_task(1, 1, first=True)

    def pairs(tp, _):
        t = tp * 2
        do_task(t, 0)
        do_task(t + 1, 1)
        return 0

    lax.fori_loop(1, TPW // 2 - 1, pairs, 0)
    do_task(TPW - 2, 0)
    do_task(TPW - 1, 1, fire_next=False)

    pltpu.make_async_copy(idx_hbm.at[0, pl.ds(0, K)], idx0, isem0).wait()
    for p in (0, 1):
        tb_v, ssem = bufs[p][4], bufs[p][5]
        pltpu.make_async_copy(
            tb_v, out_hbm.at[pl.ds(0, DR), pl.ds(0, K)], ssem).wait()


@jax.jit
def _emb_gather(idx3d, table):
    run = functools.partial(
        pl.kernel,
        out_type=jax.ShapeDtypeStruct((L * DR, B // IDX_W, 8, IDX_W),
                                      jnp.float32),
        mesh=plsc.VectorSubcoreMesh(core_axis_name="c", subcore_axis_name="s"),
        scratch_types=[
            pltpu.VMEM((K, IDX_W), jnp.int32),
            pltpu.VMEM((K, IDX_W), jnp.int32),
            pltpu.VMEM((CH, D), jnp.float32),
            pltpu.VMEM((CH, D), jnp.float32),
            pltpu.VMEM((DR, K, 8, IDX_W), jnp.float32),
            pltpu.VMEM((DR, K, 8, IDX_W), jnp.float32),
            pltpu.SemaphoreType.DMA,
            pltpu.SemaphoreType.DMA,
            pltpu.SemaphoreType.DMA,
            pltpu.SemaphoreType.DMA,
            pltpu.SemaphoreType.DMA,
            pltpu.SemaphoreType.DMA,
        ],
        compiler_params=pltpu.CompilerParams(
            use_tc_tiling_on_sc=False, needs_layout_passes=False),
    )(_emb_gather_body)
    return run(idx3d, table)


def kernel(indices, table):
    idx3d = jnp.transpose(indices).reshape(L, B // IDX_W, IDX_W).astype(jnp.int32)
    tiles = _emb_gather(idx3d, table)
    out5 = tiles.reshape(L, DR, B // IDX_W, 8, IDX_W)
    return out5.transpose(0, 2, 4, 1, 3).reshape(L, B, D)


# speedup vs baseline: 1.5576x; 1.5576x over previous
import functools

import jax
import jax.numpy as jnp
from jax import lax
from jax.experimental import pallas as pl
from jax.experimental.pallas import tpu as pltpu
from jax.experimental.pallas import tpu_sc as plsc

B = 4096
L = 200
D = 32
N = B * L

IDX_W = 128
K = 4
CH = K * IDX_W
DR = D // 8

_info = plsc.get_sparse_core_info()
NC, NS = _info.num_cores, _info.num_subcores
NW = NC * NS
NTASK = N // CH
TPW = NTASK // NW
BCHUNKS = B // CH
assert NTASK % NW == 0 and TPW % 2 == 0


def _emb_gather_body(idx_hbm, table_hbm, out_hbm,
                     idx0, idx1, rows0, rows1, tb0, tb1,
                     isem0, isem1, gsem0, gsem1, ssem0, ssem1):
    wid = lax.axis_index("s") * NC + lax.axis_index("c")
    g0 = wid * TPW
    bufs = ((idx0, isem0, rows0, gsem0, tb0, ssem0),
            (idx1, isem1, rows1, gsem1, tb1, ssem1))

    iota16 = lax.iota(jnp.int32, 16)
    dcols = [jnp.full((16,), d, jnp.int32) for d in range(D)]

    def task_lb(t):
        g = g0 + t
        return g // BCHUNKS, (g % BCHUNKS) * K

    def start_idx(t, p):
        idx_v, isem = bufs[p][0], bufs[p][1]
        l, bc0 = task_lb(t)
        pltpu.make_async_copy(
            idx_hbm.at[l, pl.ds(pl.multiple_of(bc0, K), K)], idx_v, isem).start()

    def fire_gathers(p):
        idx_v, _, rows_v, gsem = bufs[p][:4]
        gathers = [
            pltpu.make_async_copy(
                table_hbm.at[idx_v.at[j]],
                rows_v.at[pl.ds(j * IDX_W, IDX_W)],
                gsem,
            )
            for j in range(K)
        ]
        for g in gathers:
            g.start()

    def wait_gathers(p):
        idx_v, _, rows_v, gsem = bufs[p][:4]
        for j in range(K):
            pltpu.make_async_copy(
                table_hbm.at[idx_v.at[j]],
                rows_v.at[pl.ds(j * IDX_W, IDX_W)],
                gsem,
            ).wait()

    def do_task(t, p, first=False, fire_next=True):
        idx_v, isem, rows_v, gsem, tb_v, ssem = bufs[p]
        q = 1 - p
        l, bc0 = task_lb(t)
        wait_gathers(p)
        if fire_next:
            pltpu.make_async_copy(
                idx_hbm.at[0, pl.ds(0, K)], bufs[q][0], bufs[q][1]).wait()
            fire_gathers(q)
            start_idx(jnp.minimum(t + 2, TPW - 1), p)
        if not first:
            pltpu.make_async_copy(
                tb_v, out_hbm.at[pl.ds(0, DR), pl.ds(0, K)], ssem).wait()

        def tr_block(j, _):
            base = j * IDX_W + iota16
            for bg in range(8):
                ridx = base + bg * 16
                vs = [plsc.load_gather(rows_v, [ridx, dcols[d]])
                      for d in range(D)]
                for d in range(D):
                    tb_v[d // 8, j, d % 8, pl.ds(bg * 16, 16)] = vs[d]
            return 0

        lax.fori_loop(0, K, tr_block, 0)
        pltpu.make_async_copy(
            tb_v,
            out_hbm.at[pl.ds(pl.multiple_of(l * DR, DR), DR),
                       pl.ds(pl.multiple_of(bc0, K), K)],
            ssem).start()

    start_idx(0, 0)
    pltpu.make_async_copy(idx_hbm.at[0, pl.ds(0, K)], idx0, isem0).wait()
    fire_gathers(0)
    start_idx(1, 1)

    do_task(0, 0, first=True)
    do_task(1, 1, first=True)

    def pairs(tp, _):
        t = tp * 2
        do_task(t, 0)
        do_task(t + 1, 1)
        return 0

    lax.fori_loop(1, TPW // 2 - 1, pairs, 0)
    do_task(TPW - 2, 0)
    do_task(TPW - 1, 1, fire_next=False)

    pltpu.make_async_copy(idx_hbm.at[0, pl.ds(0, K)], idx0, isem0).wait()
    for p in (0, 1):
        tb_v, ssem = bufs[p][4], bufs[p][5]
        pltpu.make_async_copy(
            tb_v, out_hbm.at[pl.ds(0, DR), pl.ds(0, K)], ssem).wait()


@jax.jit
def _emb_gather(idx3d, table):
    run = functools.partial(
        pl.kernel,
        out_type=jax.ShapeDtypeStruct((L * DR, B // IDX_W, 8, IDX_W),
                                      jnp.float32),
        mesh=plsc.VectorSubcoreMesh(core_axis_name="c", subcore_axis_name="s"),
        scratch_types=[
            pltpu.VMEM((K, IDX_W), jnp.int32),
            pltpu.VMEM((K, IDX_W), jnp.int32),
            pltpu.VMEM((CH, D), jnp.float32),
            pltpu.VMEM((CH, D), jnp.float32),
            pltpu.VMEM((DR, K, 8, IDX_W), jnp.float32),
            pltpu.VMEM((DR, K, 8, IDX_W), jnp.float32),
            pltpu.SemaphoreType.DMA,
            pltpu.SemaphoreType.DMA,
            pltpu.SemaphoreType.DMA,
            pltpu.SemaphoreType.DMA,
            pltpu.SemaphoreType.DMA,
            pltpu.SemaphoreType.DMA,
        ],
        compiler_params=pltpu.CompilerParams(
            use_tc_tiling_on_sc=False, needs_layout_passes=False),
    )(_emb_gather_body)
    return run(idx3d, table)


def kernel(indices, table):
    idx3d = jnp.transpose(indices).reshape(L, B // IDX_W, IDX_W).astype(jnp.int32)
    tiles = _emb_gather(idx3d, table)
    out5 = tiles.reshape(L, DR, B // IDX_W, 8, IDX_W)
    return out5.transpose(0, 2, 4, 1, 3).reshape(L, B, D)
